# baseline (speedup 1.0000x reference)
import numpy as np
import jax
import jax.numpy as jnp
from jax.experimental import pallas as pl
from jax.experimental.pallas import tpu as pltpu

_H_DISC = 4
_W_DISC = 4
_DISC = _H_DISC * _W_DISC
_CH_BLOCK = 24


def _interp_matrix(n_in: int, n_out: int) -> np.ndarray:
    a = np.zeros((n_out, n_in), np.float64)
    s = n_in / n_out
    for i in range(n_out):
        x = (i + 0.5) * s - 0.5
        lo = int(np.floor(x))
        t = x - lo
        for idx, w in ((lo, 1.0 - t), (lo + 1, t)):
            a[i, min(max(idx, 0), n_in - 1)] += w
    return a.astype(np.float32)


def _upsample_kernel(emb_ref, k_ref, f_ref, o_ref):
    _, cb, fh, fw, seq = f_ref.shape
    et = emb_ref[...].T
    kt = k_ref[...]
    for c in range(cb):
        ec = et[c * _DISC:(c + 1) * _DISC, :]
        m = jax.lax.dot_general(kt, ec, (((0,), (0,)), ((), ())),
                                preferred_element_type=jnp.float32)
        o_ref[0, c] = f_ref[0, c] + m.reshape(fh, fw, seq)


def kernel(f, emb_w):
    batch, seq, ch, fh, fw = f.shape
    hw = fh * fw

    a1 = _interp_matrix(_H_DISC, fh)
    a2 = _interp_matrix(_W_DISC, fw)
    kmat = jnp.asarray(np.kron(a1, a2).T.copy())

    ft = jnp.transpose(f, (0, 2, 3, 4, 1))

    cb = _CH_BLOCK
    grid = (batch, ch // cb)

    out_t = pl.pallas_call(
        _upsample_kernel,
        grid=grid,
        in_specs=[
            pl.BlockSpec((seq, cb * _DISC), lambda b, i: (0, i)),
            pl.BlockSpec((_DISC, hw), lambda b, i: (0, 0)),
            pl.BlockSpec((1, cb, fh, fw, seq),
                         lambda b, i: (b, i, 0, 0, 0)),
        ],
        out_specs=pl.BlockSpec((1, cb, fh, fw, seq),
                               lambda b, i: (b, i, 0, 0, 0)),
        out_shape=jax.ShapeDtypeStruct((batch, ch, fh, fw, seq), jnp.float32),
        compiler_params=pltpu.CompilerParams(
            dimension_semantics=("arbitrary", "arbitrary"),
            vmem_limit_bytes=100 * 1024 * 1024,
        ),
    )(emb_w, kmat, ft)
    return jnp.transpose(out_t, (0, 4, 1, 2, 3))
